# baseline (speedup 1.0000x reference)
import functools

import jax
import jax.numpy as jnp
from jax import lax
from jax.experimental import pallas as pl
from jax.experimental.pallas import tpu as pltpu
from jax.experimental.pallas import tpu_sc as plsc

N = 10000
D = 128
DH = D // 2
P = 10112
E = 320000
C = 128
NC = 2
NS = 16
NW = NC * NS
CH = 80
E_PAD = NW * C * CH
NCHUNK = NW * CH
RPT = P // NS
NB = 5

_MESH = plsc.VectorSubcoreMesh(core_axis_name="c", subcore_axis_name="s")
_f32 = jnp.float32
_PIECES = ((0, C), (C, C), (2 * C, C), (3 * C, C), (4 * C, RPT - 4 * C))


def _fill_1d(ref, n, value):
    vec = jnp.full((16,), value, _f32)
    for j in range(n // 16):
        ref[pl.ds(j * 16, 16)] = vec
    if n % 16:
        ref[pl.ds(n - 16, 16)] = vec


@functools.partial(
    pl.kernel,
    out_type=jax.ShapeDtypeStruct((2 * P,), _f32),
    mesh=_MESH,
    scratch_types=[
        pltpu.VMEM((CH, 2, C), jnp.int32),
        pltpu.VMEM((C,), _f32),
        pltpu.VMEM((RPT,), _f32),
        pltpu.VMEM_SHARED((P,), _f32),
        pltpu.SemaphoreType.DMA,
    ],
)
def _deg_kernel(idxp_hbm, out_hbm, idx_v, ones_v, stage_v, deg_sh, sem):
    c = lax.axis_index("c")
    s = lax.axis_index("s")
    wid = s * NC + c
    lo = s * RPT
    cbase = wid * CH

    pltpu.sync_copy(idxp_hbm.at[pl.ds(cbase, CH)], idx_v)

    _fill_1d(ones_v, C, 1.0)
    _fill_1d(stage_v, RPT, 0.0)

    @pl.when(c == 0)
    def _():
        _fill_1d(stage_v, RPT, 1.0)

    pltpu.sync_copy(stage_v, deg_sh.at[pl.ds(lo, RPT)])
    plsc.subcore_barrier()

    def body(m, carry):
        pltpu.async_copy(ones_v, deg_sh.at[idx_v.at[m, 1]], sem, add=True)
        return carry

    lax.fori_loop(0, CH, body, 0)

    def drain(m, carry):
        pltpu.make_async_copy(ones_v, deg_sh.at[idx_v.at[0, 1]], sem).wait()
        return carry

    lax.fori_loop(0, CH, drain, 0)
    plsc.subcore_barrier()

    pltpu.sync_copy(deg_sh.at[pl.ds(lo, RPT)], stage_v)
    pltpu.sync_copy(stage_v, out_hbm.at[pl.ds(c * P + lo, RPT)])


CHA = 2 * CH


@functools.partial(
    pl.kernel,
    out_type=jax.ShapeDtypeStruct((2 * P, DH), _f32),
    mesh=_MESH,
    scratch_types=(
        [pltpu.VMEM((CHA, 2, C), jnp.int32)]
        + [pltpu.VMEM((C, DH), _f32) for _ in range(NB)]
        + [pltpu.VMEM_SHARED((P, DH), _f32)]
        + [pltpu.SemaphoreType.DMA for _ in range(2 * NB)]
    ),
    compiler_params=pltpu.CompilerParams(use_tc_tiling_on_sc=False),
)
def _agg1_kernel(idxp_hbm, ub_hbm, out_hbm, idx_v, *bufs):
    rows = bufs[:NB]
    acc_sh = bufs[NB]
    gsem = bufs[NB + 1: 2 * NB + 1]
    ssem = bufs[2 * NB + 1:]

    c = lax.axis_index("c")
    s = lax.axis_index("s")
    lo = s * RPT
    cbase = s * CHA
    u_hbm = ub_hbm.at[c]

    pltpu.sync_copy(idxp_hbm.at[pl.ds(cbase, CHA)], idx_v)

    for off, ln in _PIECES:
        pltpu.sync_copy(u_hbm.at[pl.ds(lo + off, ln)],
                        rows[0].at[pl.ds(0, ln)])
        pltpu.sync_copy(rows[0].at[pl.ds(0, ln)],
                        acc_sh.at[pl.ds(lo + off, ln)])

    plsc.subcore_barrier()

    pltpu.async_copy(u_hbm.at[idx_v.at[0, 0]], rows[0], gsem[0])
    pltpu.async_copy(u_hbm.at[idx_v.at[1, 0]], rows[1], gsem[1])
    pltpu.async_copy(u_hbm.at[idx_v.at[2, 0]], rows[2], gsem[2])

    def body(i, carry):
        k = i * NB
        for b in range(NB):
            m = k + b
            bn = (b + 3) % NB
            pltpu.make_async_copy(
                u_hbm.at[pl.ds(0, C)], rows[b], gsem[b]).wait()
            pltpu.async_copy(
                rows[b], acc_sh.at[idx_v.at[m, 1]], ssem[b], add=True)

            @pl.when(m + 3 < CHA)
            def _():
                @pl.when(m >= 2)
                def _():
                    pltpu.make_async_copy(
                        rows[bn], acc_sh.at[idx_v.at[0, 1]],
                        ssem[bn]).wait()

                pltpu.async_copy(
                    u_hbm.at[idx_v.at[m + 3, 0]], rows[bn], gsem[bn])
        return carry

    lax.fori_loop(0, CHA // NB, body, 0)
    for b in range(NB):
        pltpu.make_async_copy(
            rows[b], acc_sh.at[idx_v.at[0, 1]], ssem[b]).wait()
    plsc.subcore_barrier()

    for off, ln in _PIECES:
        pltpu.sync_copy(acc_sh.at[pl.ds(lo + off, ln)],
                        rows[0].at[pl.ds(0, ln)])
        pltpu.sync_copy(rows[0].at[pl.ds(0, ln)],
                        out_hbm.at[pl.ds(c * P + lo + off, ln)])


@functools.partial(
    pl.kernel,
    out_type=jax.ShapeDtypeStruct((P,), _f32),
    mesh=_MESH,
    scratch_types=[
        pltpu.VMEM((CHA, 2, C), jnp.int32),
        pltpu.VMEM((C,), _f32),
        pltpu.VMEM((C,), _f32),
        pltpu.VMEM((P,), _f32),
        pltpu.VMEM((RPT,), _f32),
        pltpu.VMEM((RPT,), _f32),
        pltpu.VMEM((16,), _f32),
        pltpu.VMEM_SHARED((P,), _f32),
        pltpu.SemaphoreType.DMA,
        pltpu.SemaphoreType.DMA,
    ],
    compiler_params=pltpu.CompilerParams(needs_layout_passes=False),
)
def _agg2_kernel(idxp_hbm, u2_hbm, dinv_hbm, b2_hbm, out_hbm,
                 idx_v, vals0, vals1, u2_v, stage_v, dinv_v, b2_v,
                 acc_sh, s0, s1):
    c = lax.axis_index("c")
    s = lax.axis_index("s")
    lo = s * RPT
    cbase = s * CHA

    @pl.when(c == 0)
    def _():
        pltpu.sync_copy(u2_hbm, u2_v)
        pltpu.sync_copy(idxp_hbm.at[pl.ds(cbase, CHA)], idx_v)
        pltpu.sync_copy(dinv_hbm.at[pl.ds(lo, RPT)], dinv_v)
        pltpu.sync_copy(b2_hbm, b2_v)

        pltpu.sync_copy(u2_v.at[pl.ds(lo, RPT)], acc_sh.at[pl.ds(lo, RPT)])
        plsc.subcore_barrier()

        vals = (vals0, vals1)
        ssem = (s0, s1)

        def body(i, carry):
            k = i * 2
            for b in range(2):
                m = k + b
                for j in range(C // 16):
                    sv = idx_v[m, 0, pl.ds(j * 16, 16)]
                    vals[b][pl.ds(j * 16, 16)] = plsc.load_gather(u2_v, [sv])

                @pl.when(m >= 2)
                def _():
                    pltpu.make_async_copy(
                        vals[b], acc_sh.at[idx_v.at[0, 1]], ssem[b]).wait()

                pltpu.async_copy(
                    vals[b], acc_sh.at[idx_v.at[m, 1]], ssem[b], add=True)
            return carry

        lax.fori_loop(0, CHA // 2, body, 0)
        for b in range(2):
            pltpu.make_async_copy(
                vals[b], acc_sh.at[idx_v.at[0, 1]], ssem[b]).wait()
        plsc.subcore_barrier()

        pltpu.sync_copy(acc_sh.at[pl.ds(lo, RPT)], stage_v)
        bv = b2_v[...]
        tl = RPT - 16
        vt = dinv_v[pl.ds(tl, 16)] * stage_v[pl.ds(tl, 16)] + bv
        tail = 1.0 / (1.0 + jnp.exp(-vt))
        for off in range(0, RPT - 16, 16):
            v = dinv_v[pl.ds(off, 16)] * stage_v[pl.ds(off, 16)] + bv
            stage_v[pl.ds(off, 16)] = 1.0 / (1.0 + jnp.exp(-v))
        stage_v[pl.ds(tl, 16)] = tail
        pltpu.sync_copy(stage_v, out_hbm.at[pl.ds(lo, RPT)])


def _mm_body(x_ref, w_ref, h_ref):
    h_ref[...] = jnp.dot(x_ref[...], w_ref[...], preferred_element_type=_f32)


def _scale_body(h_ref, deg_ref, ub_ref, dinv_ref):
    deg = deg_ref[0, :] + deg_ref[1, :]
    dinv = lax.rsqrt(deg)
    u = dinv[:, None] * h_ref[...]
    ub_ref[0] = u[:, :DH]
    ub_ref[1] = u[:, DH:]
    dinv_ref[...] = dinv


def _relu_mv_body(p_ref, dinv_ref, b1_ref, w2_ref, u2_ref):
    ssum = jnp.concatenate([p_ref[0], p_ref[1]], axis=1)
    dinv = dinv_ref[...]
    h = jnp.maximum(dinv[:, None] * ssum + b1_ref[...][None, :], 0.0)
    z = jnp.dot(h, w2_ref[...], preferred_element_type=_f32)
    u2_ref[...] = dinv * z[:, 0]


def kernel(x, edge_index, W1, b1, W2, b2):
    src = edge_index[0].astype(jnp.int32)
    dst = edge_index[1].astype(jnp.int32)

    npad = E_PAD - E
    pad_idx = (N + (jnp.arange(npad, dtype=jnp.int32) % (P - N)))
    src_p = jnp.concatenate([src, pad_idx])
    dst_p = jnp.concatenate([dst, pad_idx])
    idxp = jnp.stack(
        [src_p.reshape(NCHUNK, C), dst_p.reshape(NCHUNK, C)], axis=1)

    x_p = jnp.pad(x, ((0, P - N), (0, 0)))

    deg_p = _deg_kernel(idxp).reshape(2, P)
    h1 = pl.pallas_call(
        _mm_body,
        out_shape=jax.ShapeDtypeStruct((P, D), _f32),
    )(x_p, W1)

    u_both, dinv = pl.pallas_call(
        _scale_body,
        out_shape=(jax.ShapeDtypeStruct((2, P, DH), _f32),
                   jax.ShapeDtypeStruct((P,), _f32)),
    )(h1, deg_p)

    p_both = _agg1_kernel(idxp, u_both)

    u2 = pl.pallas_call(
        _relu_mv_body,
        out_shape=jax.ShapeDtypeStruct((P,), _f32),
    )(p_both.reshape(2, P, DH), dinv, b1, W2)

    b2_16 = jnp.full((16,), b2[0], _f32)
    out_pad = _agg2_kernel(idxp, u2, dinv, b2_16)

    return out_pad[:N].reshape(N, 1)


# speedup vs baseline: 1.0188x; 1.0188x over previous
import functools

import jax
import jax.numpy as jnp
from jax import lax
from jax.experimental import pallas as pl
from jax.experimental.pallas import tpu as pltpu
from jax.experimental.pallas import tpu_sc as plsc

N = 10000
D = 128
DH = D // 2
P = 10112
E = 320000
C = 128
NC = 2
NS = 16
NW = NC * NS
CH = 80
E_PAD = NW * C * CH
NCHUNK = NW * CH
RPT = P // NS
NB = 5

_MESH = plsc.VectorSubcoreMesh(core_axis_name="c", subcore_axis_name="s")
_f32 = jnp.float32
_PIECES = ((0, C), (C, C), (2 * C, C), (3 * C, C), (4 * C, RPT - 4 * C))


def _fill_1d(ref, n, value):
    vec = jnp.full((16,), value, _f32)
    for j in range(n // 16):
        ref[pl.ds(j * 16, 16)] = vec
    if n % 16:
        ref[pl.ds(n - 16, 16)] = vec


@functools.partial(
    pl.kernel,
    out_type=jax.ShapeDtypeStruct((2 * P,), _f32),
    mesh=_MESH,
    scratch_types=[
        pltpu.VMEM((CH, 2, C), jnp.int32),
        pltpu.VMEM((C,), _f32),
        pltpu.VMEM((RPT,), _f32),
        pltpu.VMEM_SHARED((P,), _f32),
        pltpu.SemaphoreType.DMA,
    ],
)
def _deg_kernel(idxp_hbm, out_hbm, idx_v, ones_v, stage_v, deg_sh, sem):
    c = lax.axis_index("c")
    s = lax.axis_index("s")
    wid = s * NC + c
    lo = s * RPT
    cbase = wid * CH

    pltpu.sync_copy(idxp_hbm.at[pl.ds(cbase, CH)], idx_v)

    _fill_1d(ones_v, C, 1.0)
    _fill_1d(stage_v, RPT, 0.0)

    @pl.when(c == 0)
    def _():
        _fill_1d(stage_v, RPT, 1.0)

    pltpu.sync_copy(stage_v, deg_sh.at[pl.ds(lo, RPT)])
    plsc.subcore_barrier()

    def body(m, carry):
        pltpu.async_copy(ones_v, deg_sh.at[idx_v.at[m, 1]], sem, add=True)
        return carry

    lax.fori_loop(0, CH, body, 0)

    def drain(m, carry):
        pltpu.make_async_copy(ones_v, deg_sh.at[idx_v.at[0, 1]], sem).wait()
        return carry

    lax.fori_loop(0, CH, drain, 0)
    plsc.subcore_barrier()

    pltpu.sync_copy(deg_sh.at[pl.ds(lo, RPT)], stage_v)
    pltpu.sync_copy(stage_v, out_hbm.at[pl.ds(c * P + lo, RPT)])


CHA = 2 * CH


@functools.partial(
    pl.kernel,
    out_type=jax.ShapeDtypeStruct((2 * P, DH), _f32),
    mesh=_MESH,
    scratch_types=(
        [pltpu.VMEM((CHA, 2, C), jnp.int32)]
        + [pltpu.VMEM((C, DH), _f32) for _ in range(NB)]
        + [pltpu.VMEM_SHARED((P, DH), _f32)]
        + [pltpu.SemaphoreType.DMA for _ in range(2 * NB)]
    ),
    compiler_params=pltpu.CompilerParams(use_tc_tiling_on_sc=False),
)
def _agg1_kernel(idxp_hbm, ub_hbm, out_hbm, idx_v, *bufs):
    rows = bufs[:NB]
    acc_sh = bufs[NB]
    gsem = bufs[NB + 1: 2 * NB + 1]
    ssem = bufs[2 * NB + 1:]

    c = lax.axis_index("c")
    s = lax.axis_index("s")
    lo = s * RPT
    cbase = s * CHA
    u_hbm = ub_hbm.at[c]

    pltpu.sync_copy(idxp_hbm.at[pl.ds(cbase, CHA)], idx_v)

    for off, ln in _PIECES:
        pltpu.sync_copy(u_hbm.at[pl.ds(lo + off, ln)],
                        rows[0].at[pl.ds(0, ln)])
        pltpu.sync_copy(rows[0].at[pl.ds(0, ln)],
                        acc_sh.at[pl.ds(lo + off, ln)])

    plsc.subcore_barrier()

    pltpu.async_copy(u_hbm.at[idx_v.at[0, 0]], rows[0], gsem[0])
    pltpu.async_copy(u_hbm.at[idx_v.at[1, 0]], rows[1], gsem[1])
    pltpu.async_copy(u_hbm.at[idx_v.at[2, 0]], rows[2], gsem[2])

    def body(i, carry):
        k = i * NB
        for b in range(NB):
            m = k + b
            bn = (b + 3) % NB
            pltpu.make_async_copy(
                u_hbm.at[pl.ds(0, C)], rows[b], gsem[b]).wait()
            pltpu.async_copy(
                rows[b], acc_sh.at[idx_v.at[m, 1]], ssem[b], add=True)

            @pl.when(m + 3 < CHA)
            def _():
                @pl.when(m >= 2)
                def _():
                    pltpu.make_async_copy(
                        rows[bn], acc_sh.at[idx_v.at[0, 1]],
                        ssem[bn]).wait()

                pltpu.async_copy(
                    u_hbm.at[idx_v.at[m + 3, 0]], rows[bn], gsem[bn])
        return carry

    lax.fori_loop(0, CHA // NB, body, 0)
    for b in range(NB):
        pltpu.make_async_copy(
            rows[b], acc_sh.at[idx_v.at[0, 1]], ssem[b]).wait()
    plsc.subcore_barrier()

    for off, ln in _PIECES:
        pltpu.sync_copy(acc_sh.at[pl.ds(lo + off, ln)],
                        rows[0].at[pl.ds(0, ln)])
        pltpu.sync_copy(rows[0].at[pl.ds(0, ln)],
                        out_hbm.at[pl.ds(c * P + lo + off, ln)])


@functools.partial(
    pl.kernel,
    out_type=jax.ShapeDtypeStruct((2 * P,), _f32),
    mesh=_MESH,
    scratch_types=[
        pltpu.VMEM((CH, 2, C), jnp.int32),
        pltpu.VMEM((C,), _f32),
        pltpu.VMEM((C,), _f32),
        pltpu.VMEM((P,), _f32),
        pltpu.VMEM((RPT,), _f32),
        pltpu.VMEM_SHARED((P,), _f32),
        pltpu.SemaphoreType.DMA,
        pltpu.SemaphoreType.DMA,
    ],
    compiler_params=pltpu.CompilerParams(needs_layout_passes=False),
)
def _agg2_kernel(idxp_hbm, u2_hbm, out_hbm,
                 idx_v, vals0, vals1, u2_v, stage_v, acc_sh, s0, s1):
    c = lax.axis_index("c")
    s = lax.axis_index("s")
    wid = s * NC + c
    lo = s * RPT
    cbase = wid * CH

    pltpu.sync_copy(u2_hbm, u2_v)
    pltpu.sync_copy(idxp_hbm.at[pl.ds(cbase, CH)], idx_v)

    @pl.when(c == 0)
    def _():
        pltpu.sync_copy(u2_v.at[pl.ds(lo, RPT)], acc_sh.at[pl.ds(lo, RPT)])

    @pl.when(c == 1)
    def _():
        _fill_1d(stage_v, RPT, 0.0)
        pltpu.sync_copy(stage_v, acc_sh.at[pl.ds(lo, RPT)])

    plsc.subcore_barrier()

    vals = (vals0, vals1)
    ssem = (s0, s1)

    def body(i, carry):
        k = i * 2
        for b in range(2):
            m = k + b
            for j in range(C // 16):
                sv = idx_v[m, 0, pl.ds(j * 16, 16)]
                vals[b][pl.ds(j * 16, 16)] = plsc.load_gather(u2_v, [sv])

            @pl.when(m >= 2)
            def _():
                pltpu.make_async_copy(
                    vals[b], acc_sh.at[idx_v.at[0, 1]], ssem[b]).wait()

            pltpu.async_copy(
                vals[b], acc_sh.at[idx_v.at[m, 1]], ssem[b], add=True)
        return carry

    lax.fori_loop(0, CH // 2, body, 0)
    for b in range(2):
        pltpu.make_async_copy(vals[b], acc_sh.at[idx_v.at[0, 1]], ssem[b]).wait()
    plsc.subcore_barrier()

    pltpu.sync_copy(acc_sh.at[pl.ds(lo, RPT)], stage_v)
    pltpu.sync_copy(stage_v, out_hbm.at[pl.ds(c * P + lo, RPT)])


def _mm_body(x_ref, w_ref, h_ref):
    h_ref[...] = jnp.dot(x_ref[...], w_ref[...], preferred_element_type=_f32)


def _scale_body(h_ref, deg_ref, ub_ref, dinv_ref):
    deg = deg_ref[0, :] + deg_ref[1, :]
    dinv = lax.rsqrt(deg)
    u = dinv[:, None] * h_ref[...]
    ub_ref[0] = u[:, :DH]
    ub_ref[1] = u[:, DH:]
    dinv_ref[...] = dinv


def _relu_mv_body(p_ref, dinv_ref, b1_ref, w2_ref, u2_ref):
    ssum = jnp.concatenate([p_ref[0], p_ref[1]], axis=1)
    dinv = dinv_ref[...]
    h = jnp.maximum(dinv[:, None] * ssum + b1_ref[...][None, :], 0.0)
    z = jnp.dot(h, w2_ref[...], preferred_element_type=_f32)
    u2_ref[...] = dinv * z[:, 0]


def _final_body(q_ref, dinv_ref, b2_ref, out_ref):
    v = dinv_ref[...] * (q_ref[0] + q_ref[1]) + b2_ref[0]
    out_ref[...] = jax.nn.sigmoid(v)


def kernel(x, edge_index, W1, b1, W2, b2):
    src = edge_index[0].astype(jnp.int32)
    dst = edge_index[1].astype(jnp.int32)

    npad = E_PAD - E
    pad_idx = (N + (jnp.arange(npad, dtype=jnp.int32) % (P - N)))
    src_p = jnp.concatenate([src, pad_idx])
    dst_p = jnp.concatenate([dst, pad_idx])
    idxp = jnp.stack(
        [src_p.reshape(NCHUNK, C), dst_p.reshape(NCHUNK, C)], axis=1)

    x_p = jnp.pad(x, ((0, P - N), (0, 0)))

    deg_p = _deg_kernel(idxp).reshape(2, P)
    h1 = pl.pallas_call(
        _mm_body,
        out_shape=jax.ShapeDtypeStruct((P, D), _f32),
    )(x_p, W1)

    u_both, dinv = pl.pallas_call(
        _scale_body,
        out_shape=(jax.ShapeDtypeStruct((2, P, DH), _f32),
                   jax.ShapeDtypeStruct((P,), _f32)),
    )(h1, deg_p)

    p_both = _agg1_kernel(idxp, u_both)

    u2 = pl.pallas_call(
        _relu_mv_body,
        out_shape=jax.ShapeDtypeStruct((P,), _f32),
    )(p_both.reshape(2, P, DH), dinv, b1, W2)

    part2 = _agg2_kernel(idxp, u2).reshape(2, P)

    out_pad = pl.pallas_call(
        _final_body,
        out_shape=jax.ShapeDtypeStruct((P,), _f32),
    )(part2, dinv, b2)

    return out_pad[:N].reshape(N, 1)
